# baseline (speedup 1.0000x reference)
import functools

import jax
import jax.numpy as jnp
from jax import lax
from jax.experimental import pallas as pl
from jax.experimental.pallas import tpu as pltpu
from jax.experimental.pallas import tpu_sc as plsc

B = 4096
D = 64
VOCAB = 1000000
N_NEGS = 20

_info = plsc.get_sparse_core_info()
NC, NS, L = _info.num_cores, _info.num_subcores, _info.num_lanes
NW = NC * NS
BW = B // NW
CB = 4
CROWS = CB * N_NEGS
NCHUNK = BW // CB
NBUF = 2
NDOT = BW * N_NEGS


def _logsig(x):
    a = jnp.exp(-jnp.abs(x))
    z = a / (a + 2.0)
    z2 = z * z
    p = 1.0 + z2 * (1.0 / 3 + z2 * (1.0 / 5 + z2 * (1.0 / 7 + z2 * (1.0 / 9))))
    return jnp.minimum(x, 0.0) - 2.0 * z * p


@functools.partial(
    pl.kernel,
    out_type=jax.ShapeDtypeStruct((NW, 16), jnp.float32),
    mesh=plsc.VectorSubcoreMesh(core_axis_name="c", subcore_axis_name="s"),
    compiler_params=pltpu.CompilerParams(
        needs_layout_passes=False, use_tc_tiling_on_sc=False),
    scratch_types=[
        pltpu.VMEM((NDOT,), jnp.int32),
        pltpu.VMEM((BW, D), jnp.float32),
        pltpu.VMEM((BW, D), jnp.float32),
        pltpu.VMEM((CROWS, D), jnp.float32),
        pltpu.VMEM((CROWS, D), jnp.float32),
        pltpu.VMEM((16,), jnp.float32),
        pltpu.SemaphoreType.DMA,
        pltpu.SemaphoreType.DMA,
    ],
)
def _sgns_sc(nwords_hbm, true_hbm, outv_hbm, emb_hbm, out_hbm,
             idx_v, true_v, outv_v, rows0_v, rows1_v, acc_v, sem0, sem1):
    wid = lax.axis_index("s") * NC + lax.axis_index("c")
    bufs = (rows0_v, rows1_v)
    sems = (sem0, sem1)
    iota = lax.iota(jnp.int32, 16)

    pltpu.sync_copy(nwords_hbm.at[pl.ds(wid * NDOT, NDOT)], idx_v)
    pltpu.sync_copy(true_hbm.at[pl.ds(wid * BW, BW)], true_v)
    pltpu.sync_copy(outv_hbm.at[pl.ds(wid * BW, BW)], outv_v)

    def gather_start(c, buf, sem):
        src = emb_hbm.at[idx_v.at[pl.ds(c * CROWS, CROWS)]]
        pltpu.make_async_copy(src, buf, sem).start()

    def gather_wait(buf, sem):
        src = emb_hbm.at[idx_v.at[pl.ds(0, CROWS)]]
        pltpu.make_async_copy(src, buf, sem).wait()

    def compute_chunk(c, rows, acc):
        dvec = jnp.zeros((16,), jnp.float32)
        cnt = 0
        for bi in range(CB):
            bl = c * CB + bi
            t0 = true_v[bl, pl.ds(0, 16)]
            t1 = true_v[bl, pl.ds(16, 16)]
            t2 = true_v[bl, pl.ds(32, 16)]
            t3 = true_v[bl, pl.ds(48, 16)]
            for j in range(N_NEGS):
                r = bi * N_NEGS + j
                e0 = rows[r, pl.ds(0, 16)]
                e1 = rows[r, pl.ds(16, 16)]
                e2 = rows[r, pl.ds(32, 16)]
                e3 = rows[r, pl.ds(48, 16)]
                dot = jnp.sum(e0 * t0 + e1 * t1 + e2 * t2 + e3 * t3)
                dvec = jnp.where(iota == (cnt % 16), dot, dvec)
                cnt += 1
                if cnt % 16 == 0:
                    acc = acc + _logsig(-dvec)
        return acc

    for s in range(NBUF):
        gather_start(s, bufs[s], sems[s])

    def ring_body(i, acc):
        for s in range(NBUF):
            c = i * NBUF + s
            gather_wait(bufs[s], sems[s])
            acc = compute_chunk(c, bufs[s], acc)

            @pl.when(c + NBUF < NCHUNK)
            def _():
                gather_start(c + NBUF, bufs[s], sems[s])
        return acc

    acc = lax.fori_loop(0, NCHUNK // NBUF, ring_body,
                        jnp.zeros((16,), jnp.float32))

    def o_body(g, acc):
        dvec = jnp.zeros((16,), jnp.float32)
        for i in range(16):
            bl = g * 16 + i
            t0 = true_v[bl, pl.ds(0, 16)]
            t1 = true_v[bl, pl.ds(16, 16)]
            t2 = true_v[bl, pl.ds(32, 16)]
            t3 = true_v[bl, pl.ds(48, 16)]
            o0 = outv_v[bl, pl.ds(0, 16)]
            o1 = outv_v[bl, pl.ds(16, 16)]
            o2 = outv_v[bl, pl.ds(32, 16)]
            o3 = outv_v[bl, pl.ds(48, 16)]
            dot = jnp.sum(o0 * t0 + o1 * t1 + o2 * t2 + o3 * t3)
            dvec = jnp.where(iota == i, dot, dvec)
        return acc + _logsig(dvec)

    acc = lax.fori_loop(0, BW // 16, o_body, acc)
    acc_v[...] = acc
    pltpu.sync_copy(acc_v, out_hbm.at[wid])


def kernel(true_vecs, out_vecs, emb_table):
    nwords = jax.random.randint(
        jax.random.key(42), (B, N_NEGS), 0, VOCAB).reshape(-1)
    tv = true_vecs.reshape(B, D)
    ov = out_vecs.reshape(B, D)
    partials = _sgns_sc(nwords, tv, ov, emb_table)
    return -(jnp.sum(partials) / jnp.float32(B))
